# baseline (speedup 1.0000x reference)
import functools

import jax
import jax.numpy as jnp
from jax import lax
from jax.experimental import pallas as pl
from jax.experimental.pallas import tpu as pltpu
from jax.experimental.pallas import tpu_sc as plsc

N_AGENT = 10000
N_POI = 10000
D = 256
NC = 2
NS = 16
NW = NC * NS
B1 = 128
B2 = 80
AW = 144


def _mm_body(x_ref, w_ref, o_ref):
    o_ref[...] = jnp.dot(x_ref[...], w_ref[...], preferred_element_type=jnp.float32)


def _mm(x, w, bm=2048):
    M, K = x.shape
    _, N = w.shape
    return pl.pallas_call(
        _mm_body,
        grid=(pl.cdiv(M, bm),),
        in_specs=[pl.BlockSpec((bm, K), lambda i: (i, 0)),
                  pl.BlockSpec((K, N), lambda i: (0, 0))],
        out_specs=pl.BlockSpec((bm, N), lambda i: (i, 0)),
        out_shape=jax.ShapeDtypeStruct((M, N), jnp.float32),
    )(x, w)


def _mm_bias_body(x_ref, w_ref, b_ref, o_ref):
    o_ref[...] = jnp.dot(x_ref[...], w_ref[...],
                         preferred_element_type=jnp.float32) + b_ref[...]


def _mm_bias(x, w, b, bm=2048):
    M, K = x.shape
    _, N = w.shape
    return pl.pallas_call(
        _mm_bias_body,
        grid=(pl.cdiv(M, bm),),
        in_specs=[pl.BlockSpec((bm, K), lambda i: (i, 0)),
                  pl.BlockSpec((K, N), lambda i: (0, 0)),
                  pl.BlockSpec((1, N), lambda i: (0, 0))],
        out_specs=pl.BlockSpec((bm, N), lambda i: (i, 0)),
        out_shape=jax.ShapeDtypeStruct((M, N), jnp.float32),
    )(x, w, b.reshape(1, N))


def _proj_body(x_ref, w_ref, xl_ref, xa_ref):
    xl = jnp.dot(x_ref[...], w_ref[...], preferred_element_type=jnp.float32)
    xl_ref[...] = xl
    bm = xl.shape[0]
    col = lax.broadcasted_iota(jnp.int32, (bm, 16), 1)
    tail = jnp.where(col == 0, 1.0, 0.0).astype(jnp.float32)
    xa_ref[0] = jnp.concatenate([xl[:, :128], tail], axis=1)
    xa_ref[1] = jnp.concatenate([xl[:, 128:], tail], axis=1)


def _proj(x, w, bm=2048):
    M, K = x.shape
    _, N = w.shape
    return pl.pallas_call(
        _proj_body,
        grid=(pl.cdiv(M, bm),),
        in_specs=[pl.BlockSpec((bm, K), lambda i: (i, 0)),
                  pl.BlockSpec((K, N), lambda i: (0, 0))],
        out_specs=[pl.BlockSpec((bm, N), lambda i: (i, 0)),
                   pl.BlockSpec((2, bm, AW), lambda i: (0, i, 0))],
        out_shape=[jax.ShapeDtypeStruct((M, N), jnp.float32),
                   jax.ShapeDtypeStruct((2, M, AW), jnp.float32)],
    )(x, w)


def _epi_body(t0_ref, t1_ref, b_ref, h_ref):
    den = t0_ref[:, 128:129] + 1e-16
    num = jnp.concatenate([t0_ref[:, :128], t1_ref[:, :128]], axis=1)
    h_ref[...] = jnp.maximum(num / den + b_ref[...], 0.0)


def _epi(t0, t1, b, bm=2048):
    M = t0.shape[0]
    return pl.pallas_call(
        _epi_body,
        grid=(pl.cdiv(M, bm),),
        in_specs=[pl.BlockSpec((bm, AW), lambda i: (i, 0)),
                  pl.BlockSpec((bm, AW), lambda i: (i, 0)),
                  pl.BlockSpec((1, D), lambda i: (0, 0))],
        out_specs=pl.BlockSpec((bm, D), lambda i: (i, 0)),
        out_shape=jax.ShapeDtypeStruct((M, D), jnp.float32),
    )(t0, t1, b.reshape(1, D))


def _score_call(xl, xr, src, dst, att, eat, interpret=False):
    E = src.shape[0]
    n_e = E // NW
    nchunk = pl.cdiv(n_e, B1)
    last = n_e - B1
    has_eat = eat is not None
    mesh = plsc.VectorSubcoreMesh(core_axis_name="c", subcore_axis_name="s", num_cores=NC, num_subcores=NS)
    scratch = [
        pltpu.VMEM((B1,), jnp.int32),
        pltpu.VMEM((B1,), jnp.int32),
        pltpu.VMEM((B1, D), jnp.float32),
        pltpu.VMEM((B1, D), jnp.float32),
        pltpu.VMEM((D,), jnp.float32),
        pltpu.VMEM((B1,), jnp.float32),
        pltpu.SemaphoreType.DMA,
    ]
    if has_eat:
        scratch.insert(4, pltpu.VMEM((B1, D), jnp.float32))

    @functools.partial(pl.kernel,
                       out_type=jax.ShapeDtypeStruct((E,), jnp.float32),
                       mesh=mesh, scratch_types=scratch, interpret=interpret,
                       compiler_params=pltpu.CompilerParams(use_tc_tiling_on_sc=False, needs_layout_passes=False))
    def k(*refs):
        if has_eat:
            (xl_hbm, xr_hbm, src_hbm, dst_hbm, att_hbm, eat_hbm, ex_hbm,
             si, di, xlv, xrv, ev, av, exv, sem) = refs
        else:
            (xl_hbm, xr_hbm, src_hbm, dst_hbm, att_hbm, ex_hbm,
             si, di, xlv, xrv, av, exv, sem) = refs
        wid = lax.axis_index("s") * NC + lax.axis_index("c")
        wbase = wid * n_e
        pltpu.sync_copy(att_hbm, av)
        iota = lax.iota(jnp.int32, 16)

        def chunk(c, carry):
            base = wbase + jnp.minimum(c * B1, last)
            pltpu.sync_copy(src_hbm.at[pl.ds(base, B1)], si)
            pltpu.sync_copy(dst_hbm.at[pl.ds(base, B1)], di)
            pltpu.async_copy(xl_hbm.at[si], xlv, sem).wait()
            pltpu.async_copy(xr_hbm.at[di], xrv, sem).wait()
            if has_eat:
                pltpu.sync_copy(eat_hbm.at[pl.ds(base, B1)], ev)
            for g in range(B1 // 16):
                rows = g * 16 + iota

                def hstep(hb, acc):
                    hbase = hb * 16
                    a_vec = av[pl.ds(hbase, 16)]
                    cb = jnp.full((16,), hbase, jnp.int32)
                    for kk in range(16):
                        colv = cb + kk
                        m = (plsc.load_gather(xlv, [rows, colv])
                             + plsc.load_gather(xrv, [rows, colv]))
                        if has_eat:
                            m = m + plsc.load_gather(ev, [rows, colv])
                        m = jnp.maximum(m, 0.2 * m)
                        acc = acc + m * a_vec[kk]
                    return acc

                acc = lax.fori_loop(0, D // 16, hstep,
                                    jnp.zeros((16,), jnp.float32))
                exv[pl.ds(g * 16, 16)] = jnp.exp(acc)
            pltpu.sync_copy(exv, ex_hbm.at[pl.ds(base, B1)])
            return carry

        lax.fori_loop(0, nchunk, chunk, 0)

    args = [xl, xr, src, dst, att] + ([eat] if has_eat else [])
    return k(*args)


def _agg_call(xa, n_src, src, dst, ex, n_dst, interpret=False):
    E = dst.shape[0]
    n_e = E // NS
    nchunk = pl.cdiv(n_e, B2)
    last = n_e - B2
    rng = 640
    ninit = pl.cdiv(rng, B2)
    mesh = plsc.VectorSubcoreMesh(core_axis_name="c", subcore_axis_name="s", num_cores=NC, num_subcores=NS)
    out_t = jax.ShapeDtypeStruct((2, n_dst, AW), jnp.float32)
    scratch = [
        pltpu.VMEM((B2,), jnp.int32),
        pltpu.VMEM((B2,), jnp.int32),
        pltpu.VMEM((B2,), jnp.float32),
        pltpu.VMEM((B2, AW), jnp.float32),
        pltpu.VMEM_SHARED((n_dst, AW), jnp.float32),
        pltpu.SemaphoreType.DMA,
    ]

    @functools.partial(pl.kernel, out_type=out_t, mesh=mesh,
                       scratch_types=scratch, interpret=interpret,
                       compiler_params=pltpu.CompilerParams(use_tc_tiling_on_sc=False, needs_layout_passes=False))
    def k(xa_hbm, src_hbm, dst_hbm, ex_hbm, t_hbm, di, gi, exv, rows, tbl, sem):
        cid = lax.axis_index("c")
        sid = lax.axis_index("s")
        goff = cid * n_src

        def zrow(r, carry):
            for j in range(AW // 16):
                rows[r, pl.ds(j * 16, 16)] = jnp.zeros((16,), jnp.float32)
            return carry

        lax.fori_loop(0, B2, zrow, 0)

        def zc(i, carry):
            base = jnp.minimum(sid * rng + i * B2, n_dst - B2)
            pltpu.sync_copy(rows, tbl.at[pl.ds(base, B2)])
            return carry

        lax.fori_loop(0, ninit, zc, 0)
        plsc.subcore_barrier()

        sbase = sid * n_e

        def chunk(c, carry):
            base = sbase + c * B2
            pltpu.sync_copy(dst_hbm.at[pl.ds(base, B2)], di)
            pltpu.sync_copy(src_hbm.at[pl.ds(base, B2)], gi)
            pltpu.sync_copy(ex_hbm.at[pl.ds(base, B2)], exv)
            for j in range(B2 // 16):
                sl = pl.ds(j * 16, 16)
                gi[sl] = gi[sl] + goff
            pltpu.async_copy(xa_hbm.at[gi], rows, sem).wait()

            def escale(eb, carry2):
                ebase = eb * 16
                ex_vec = exv[pl.ds(ebase, 16)]
                for kk in range(16):
                    sv = ex_vec[kk]
                    for j in range(AW // 16):
                        sl = pl.ds(j * 16, 16)
                        rows[ebase + kk, sl] = rows[ebase + kk, sl] * sv
                return carry2

            lax.fori_loop(0, B2 // 16, escale, 0)
            pltpu.sync_copy(rows, tbl.at[di], add=True)
            return carry

        lax.fori_loop(0, nchunk, chunk, 0)
        plsc.subcore_barrier()

        def oc(i, carry):
            base = jnp.minimum(sid * rng + i * B2, n_dst - B2)
            pltpu.sync_copy(tbl.at[pl.ds(base, B2)], t_hbm.at[cid, pl.ds(base, B2)])
            return carry

        lax.fori_loop(0, ninit, oc, 0)

    return k(xa, src, dst, ex)


def _gatv2_sc(x_src, x_dst, ei, Wl, Wr, att, b, n_dst, eat=None):
    xl, xa = _proj(x_src, Wl)
    xr = _mm(x_dst, Wr)
    ex = _score_call(xl, xr, ei[0], ei[1], att, eat)
    t = _agg_call(xa.reshape(2 * x_src.shape[0], AW), x_src.shape[0], ei[0], ei[1], ex, n_dst)
    return _epi(t[0], t[1], b)


def kernel(x_agent, x_poi, edge_index_spatial, edge_index_interacts, edge_attr_interacts,
           Wl11, Wr11, a11, b11, Wl12, Wr12, We12, a12, b12,
           Wl21, Wr21, a21, b21, Wl22, Wr22, We22, a22, b22,
           Wa, ba, Wp, bp):
    eat12 = _mm(edge_attr_interacts, We12)
    eat22 = _mm(edge_attr_interacts, We22)
    h_a = _gatv2_sc(x_agent, x_agent, edge_index_spatial, Wl11, Wr11, a11, b11, N_AGENT)
    h_p = _gatv2_sc(x_agent, x_poi, edge_index_interacts, Wl12, Wr12, a12, b12, N_POI, eat12)
    h_a2 = _gatv2_sc(h_a, h_a, edge_index_spatial, Wl21, Wr21, a21, b21, N_AGENT)
    h_p2 = _gatv2_sc(h_a, h_p, edge_index_interacts, Wl22, Wr22, a22, b22, N_POI, eat22)
    out_a = _mm_bias(h_a2, Wa, ba)
    out_p = _mm_bias(h_p2, Wp, bp)
    return out_a, out_p


# speedup vs baseline: 1.7955x; 1.7955x over previous
import functools

import jax
import jax.numpy as jnp
from jax import lax
from jax.experimental import pallas as pl
from jax.experimental.pallas import tpu as pltpu
from jax.experimental.pallas import tpu_sc as plsc

N_AGENT = 10000
N_POI = 10000
D = 256
NC = 2
NS = 16
NW = NC * NS
B1 = 128
B2 = 80
AW = 144


def _mm_body(x_ref, w_ref, o_ref):
    o_ref[...] = jnp.dot(x_ref[...], w_ref[...], preferred_element_type=jnp.float32)


def _mm(x, w, bm=2048):
    M, K = x.shape
    _, N = w.shape
    return pl.pallas_call(
        _mm_body,
        grid=(pl.cdiv(M, bm),),
        in_specs=[pl.BlockSpec((bm, K), lambda i: (i, 0)),
                  pl.BlockSpec((K, N), lambda i: (0, 0))],
        out_specs=pl.BlockSpec((bm, N), lambda i: (i, 0)),
        out_shape=jax.ShapeDtypeStruct((M, N), jnp.float32),
    )(x, w)


def _mm_bias_body(x_ref, w_ref, b_ref, o_ref):
    o_ref[...] = jnp.dot(x_ref[...], w_ref[...],
                         preferred_element_type=jnp.float32) + b_ref[...]


def _mm_bias(x, w, b, bm=2048):
    M, K = x.shape
    _, N = w.shape
    return pl.pallas_call(
        _mm_bias_body,
        grid=(pl.cdiv(M, bm),),
        in_specs=[pl.BlockSpec((bm, K), lambda i: (i, 0)),
                  pl.BlockSpec((K, N), lambda i: (0, 0)),
                  pl.BlockSpec((1, N), lambda i: (0, 0))],
        out_specs=pl.BlockSpec((bm, N), lambda i: (i, 0)),
        out_shape=jax.ShapeDtypeStruct((M, N), jnp.float32),
    )(x, w, b.reshape(1, N))


def _proj_body(x_ref, w_ref, xl_ref, xa_ref):
    xl = jnp.dot(x_ref[...], w_ref[...], preferred_element_type=jnp.float32)
    xl_ref[...] = xl
    bm = xl.shape[0]
    col = lax.broadcasted_iota(jnp.int32, (bm, 16), 1)
    tail = jnp.where(col == 0, 1.0, 0.0).astype(jnp.float32)
    xa_ref[0] = jnp.concatenate([xl[:, :128], tail], axis=1)
    xa_ref[1] = jnp.concatenate([xl[:, 128:], tail], axis=1)


def _proj(x, w, bm=2048):
    M, K = x.shape
    _, N = w.shape
    return pl.pallas_call(
        _proj_body,
        grid=(pl.cdiv(M, bm),),
        in_specs=[pl.BlockSpec((bm, K), lambda i: (i, 0)),
                  pl.BlockSpec((K, N), lambda i: (0, 0))],
        out_specs=[pl.BlockSpec((bm, N), lambda i: (i, 0)),
                   pl.BlockSpec((2, bm, AW), lambda i: (0, i, 0))],
        out_shape=[jax.ShapeDtypeStruct((M, N), jnp.float32),
                   jax.ShapeDtypeStruct((2, M, AW), jnp.float32)],
    )(x, w)


def _epi_body(t0_ref, t1_ref, b_ref, h_ref):
    den = t0_ref[:, 128:129] + 1e-16
    num = jnp.concatenate([t0_ref[:, :128], t1_ref[:, :128]], axis=1)
    h_ref[...] = jnp.maximum(num / den + b_ref[...], 0.0)


def _epi(t0, t1, b, bm=2048):
    M = t0.shape[0]
    return pl.pallas_call(
        _epi_body,
        grid=(pl.cdiv(M, bm),),
        in_specs=[pl.BlockSpec((bm, AW), lambda i: (i, 0)),
                  pl.BlockSpec((bm, AW), lambda i: (i, 0)),
                  pl.BlockSpec((1, D), lambda i: (0, 0))],
        out_specs=pl.BlockSpec((bm, D), lambda i: (i, 0)),
        out_shape=jax.ShapeDtypeStruct((M, D), jnp.float32),
    )(t0, t1, b.reshape(1, D))


def _lane_perm(v, idx):
    dn = lax.GatherDimensionNumbers(offset_dims=(), collapsed_slice_dims=(0,),
                                    start_index_map=(0,))
    return lax.gather(v, idx[:, None], dn, (1,),
                      mode=lax.GatherScatterMode.PROMISE_IN_BOUNDS)


def _score_call(xl, xr, src, dst, att, eat, interpret=False):
    E = src.shape[0]
    n_e = E // NW
    nchunk = pl.cdiv(n_e, B1)
    last = n_e - B1
    has_eat = eat is not None
    mesh = plsc.VectorSubcoreMesh(core_axis_name="c", subcore_axis_name="s", num_cores=NC, num_subcores=NS)
    scratch = [
        pltpu.VMEM((B1,), jnp.int32),
        pltpu.VMEM((B1,), jnp.int32),
        pltpu.VMEM((B1, D), jnp.float32),
        pltpu.VMEM((B1, D), jnp.float32),
        pltpu.VMEM((D,), jnp.float32),
        pltpu.VMEM((B1,), jnp.float32),
        pltpu.SemaphoreType.DMA,
    ]
    if has_eat:
        scratch.insert(4, pltpu.VMEM((B1, D), jnp.float32))

    @functools.partial(pl.kernel,
                       out_type=jax.ShapeDtypeStruct((E,), jnp.float32),
                       mesh=mesh, scratch_types=scratch, interpret=interpret,
                       compiler_params=pltpu.CompilerParams(use_tc_tiling_on_sc=False, needs_layout_passes=False))
    def k(*refs):
        if has_eat:
            (xl_hbm, xr_hbm, src_hbm, dst_hbm, att_hbm, eat_hbm, ex_hbm,
             si, di, xlv, xrv, ev, av, exv, sem) = refs
        else:
            (xl_hbm, xr_hbm, src_hbm, dst_hbm, att_hbm, ex_hbm,
             si, di, xlv, xrv, av, exv, sem) = refs
        wid = lax.axis_index("s") * NC + lax.axis_index("c")
        wbase = wid * n_e
        pltpu.sync_copy(att_hbm, av)
        iota = lax.iota(jnp.int32, 16)
        av16 = [av[pl.ds(kk * 16, 16)] for kk in range(D // 16)]
        rot = [(iota + sft) % 16 for sft in (1, 2, 4, 8)]

        def chunk(c, carry):
            base = wbase + jnp.minimum(c * B1, last)
            pltpu.sync_copy(src_hbm.at[pl.ds(base, B1)], si)
            pltpu.sync_copy(dst_hbm.at[pl.ds(base, B1)], di)
            pltpu.async_copy(xl_hbm.at[si], xlv, sem).wait()
            pltpu.async_copy(xr_hbm.at[di], xrv, sem).wait()
            if has_eat:
                pltpu.sync_copy(eat_hbm.at[pl.ds(base, B1)], ev)

            def grp(g, carry2):
                gbase = g * 16
                s = jnp.zeros((16,), jnp.float32)
                for e in range(16):
                    row = gbase + e
                    acc = jnp.zeros((16,), jnp.float32)
                    for kk in range(D // 16):
                        sl = pl.ds(kk * 16, 16)
                        m = xlv[row, sl] + xrv[row, sl]
                        if has_eat:
                            m = m + ev[row, sl]
                        m = jnp.maximum(m, 0.2 * m)
                        acc = acc + m * av16[kk]
                    for r in rot:
                        acc = acc + _lane_perm(acc, r)
                    s = jnp.where(iota == e, acc, s)
                exv[pl.ds(gbase, 16)] = jnp.exp(s)
                return carry2

            lax.fori_loop(0, B1 // 16, grp, 0)
            pltpu.sync_copy(exv, ex_hbm.at[pl.ds(base, B1)])
            return carry

        lax.fori_loop(0, nchunk, chunk, 0)

    args = [xl, xr, src, dst, att] + ([eat] if has_eat else [])
    return k(*args)


def _agg_call(xa, n_src, src, dst, ex, n_dst, interpret=False):
    E = dst.shape[0]
    n_e = E // NS
    nchunk = pl.cdiv(n_e, B2)
    last = n_e - B2
    rng = 640
    ninit = pl.cdiv(rng, B2)
    mesh = plsc.VectorSubcoreMesh(core_axis_name="c", subcore_axis_name="s", num_cores=NC, num_subcores=NS)
    out_t = jax.ShapeDtypeStruct((2, n_dst, AW), jnp.float32)
    scratch = [
        pltpu.VMEM((B2,), jnp.int32),
        pltpu.VMEM((B2,), jnp.int32),
        pltpu.VMEM((B2,), jnp.float32),
        pltpu.VMEM((B2, AW), jnp.float32),
        pltpu.VMEM_SHARED((n_dst, AW), jnp.float32),
        pltpu.SemaphoreType.DMA,
    ]

    @functools.partial(pl.kernel, out_type=out_t, mesh=mesh,
                       scratch_types=scratch, interpret=interpret,
                       compiler_params=pltpu.CompilerParams(use_tc_tiling_on_sc=False, needs_layout_passes=False))
    def k(xa_hbm, src_hbm, dst_hbm, ex_hbm, t_hbm, di, gi, exv, rows, tbl, sem):
        cid = lax.axis_index("c")
        sid = lax.axis_index("s")
        goff = cid * n_src

        def zrow(r, carry):
            for j in range(AW // 16):
                rows[r, pl.ds(j * 16, 16)] = jnp.zeros((16,), jnp.float32)
            return carry

        lax.fori_loop(0, B2, zrow, 0)

        def zc(i, carry):
            base = jnp.minimum(sid * rng + i * B2, n_dst - B2)
            pltpu.sync_copy(rows, tbl.at[pl.ds(base, B2)])
            return carry

        lax.fori_loop(0, ninit, zc, 0)
        plsc.subcore_barrier()

        sbase = sid * n_e

        def chunk(c, carry):
            base = sbase + c * B2
            pltpu.sync_copy(dst_hbm.at[pl.ds(base, B2)], di)
            pltpu.sync_copy(src_hbm.at[pl.ds(base, B2)], gi)
            pltpu.sync_copy(ex_hbm.at[pl.ds(base, B2)], exv)
            for j in range(B2 // 16):
                sl = pl.ds(j * 16, 16)
                gi[sl] = gi[sl] + goff
            pltpu.async_copy(xa_hbm.at[gi], rows, sem).wait()

            def escale(eb, carry2):
                ebase = eb * 16
                ex_vec = exv[pl.ds(ebase, 16)]
                for kk in range(16):
                    sv = ex_vec[kk]
                    for j in range(AW // 16):
                        sl = pl.ds(j * 16, 16)
                        rows[ebase + kk, sl] = rows[ebase + kk, sl] * sv
                return carry2

            lax.fori_loop(0, B2 // 16, escale, 0)
            pltpu.sync_copy(rows, tbl.at[di], add=True)
            return carry

        lax.fori_loop(0, nchunk, chunk, 0)
        plsc.subcore_barrier()

        def oc(i, carry):
            base = jnp.minimum(sid * rng + i * B2, n_dst - B2)
            pltpu.sync_copy(tbl.at[pl.ds(base, B2)], t_hbm.at[cid, pl.ds(base, B2)])
            return carry

        lax.fori_loop(0, ninit, oc, 0)

    return k(xa, src, dst, ex)


def _gatv2_sc(x_src, x_dst, ei, Wl, Wr, att, b, n_dst, eat=None):
    xl, xa = _proj(x_src, Wl)
    xr = _mm(x_dst, Wr)
    ex = _score_call(xl, xr, ei[0], ei[1], att, eat)
    t = _agg_call(xa.reshape(2 * x_src.shape[0], AW), x_src.shape[0], ei[0], ei[1], ex, n_dst)
    return _epi(t[0], t[1], b)


def kernel(x_agent, x_poi, edge_index_spatial, edge_index_interacts, edge_attr_interacts,
           Wl11, Wr11, a11, b11, Wl12, Wr12, We12, a12, b12,
           Wl21, Wr21, a21, b21, Wl22, Wr22, We22, a22, b22,
           Wa, ba, Wp, bp):
    eat12 = _mm(edge_attr_interacts, We12)
    eat22 = _mm(edge_attr_interacts, We22)
    h_a = _gatv2_sc(x_agent, x_agent, edge_index_spatial, Wl11, Wr11, a11, b11, N_AGENT)
    h_p = _gatv2_sc(x_agent, x_poi, edge_index_interacts, Wl12, Wr12, a12, b12, N_POI, eat12)
    h_a2 = _gatv2_sc(h_a, h_a, edge_index_spatial, Wl21, Wr21, a21, b21, N_AGENT)
    h_p2 = _gatv2_sc(h_a, h_p, edge_index_interacts, Wl22, Wr22, a22, b22, N_POI, eat22)
    out_a = _mm_bias(h_a2, Wa, ba)
    out_p = _mm_bias(h_p2, Wp, bp)
    return out_a, out_p
